# baseline (speedup 1.0000x reference)
import functools

import jax
import jax.numpy as jnp
from jax import lax
from jax.experimental import pallas as pl
from jax.experimental.pallas import tpu as pltpu
from jax.experimental.pallas import tpu_sc as plsc

VOCAB_SIZE = 1000000
EMBED_DIM = 64
BATCH = 16384
K_NEG = 20

NUM_CORES = 2
NUM_SUBCORES = 16
NUM_WORKERS = NUM_CORES * NUM_SUBCORES

B_PER_W = BATCH // NUM_WORKERS
CHUNK = 64
N_CHUNKS = B_PER_W // CHUNK
SCORES_PER_ITEM = K_NEG + 1
CHUNK_SCORES = CHUNK * SCORES_PER_ITEM
GATHER_MAX = 128


def _sc_scores_kernel(cen_w, ctx_w, neg_w, cen_emb, ctx_emb, scores_out,
                      cen_idx, pos_idx, neg_idx,
                      cen_rows, pos_rows, neg_rows, partials, scores, sem):
    wid = lax.axis_index("s") * NUM_CORES + lax.axis_index("c")
    base = wid * B_PER_W

    pltpu.sync_copy(cen_w.at[pl.ds(base, B_PER_W)], cen_idx)
    pltpu.sync_copy(ctx_w.at[pl.ds(base, B_PER_W)], pos_idx)
    pltpu.sync_copy(neg_w.at[pl.ds(base * K_NEG, B_PER_W * K_NEG)], neg_idx)

    for c in range(N_CHUNKS):
        copies = [
            pltpu.async_copy(
                cen_emb.at[cen_idx.at[pl.ds(c * CHUNK, CHUNK)]], cen_rows, sem),
            pltpu.async_copy(
                ctx_emb.at[pos_idx.at[pl.ds(c * CHUNK, CHUNK)]], pos_rows, sem),
        ]
        neg_per_chunk = CHUNK * K_NEG
        for g in range(neg_per_chunk // GATHER_MAX):
            copies.append(pltpu.async_copy(
                ctx_emb.at[neg_idx.at[pl.ds(c * neg_per_chunk + g * GATHER_MAX,
                                            GATHER_MAX)]],
                neg_rows.at[pl.ds(g * GATHER_MAX, GATHER_MAX)], sem))
        for cp in copies:
            cp.wait()

        def item_body(it, carry):
            cvec = [cen_rows[it, pl.ds(16 * j, 16)] for j in range(4)]
            acc = cvec[0] * pos_rows[it, pl.ds(0, 16)]
            for j in range(1, 4):
                acc = acc + cvec[j] * pos_rows[it, pl.ds(16 * j, 16)]
            pbase = it * SCORES_PER_ITEM * 16
            partials[pl.ds(pbase, 16)] = acc
            for k in range(K_NEG):
                r = it * K_NEG + k
                acc = cvec[0] * neg_rows[r, pl.ds(0, 16)]
                for j in range(1, 4):
                    acc = acc + cvec[j] * neg_rows[r, pl.ds(16 * j, 16)]
                partials[pl.ds(pbase + (1 + k) * 16, 16)] = -acc
            return carry

        lax.fori_loop(0, CHUNK, item_body, 0)

        ivec = lax.iota(jnp.int32, 16) * 16

        def group_body(grp, carry):
            acc = plsc.load_gather(partials, [ivec + grp * 256])
            for d in range(1, 16):
                acc = acc + plsc.load_gather(partials, [ivec + (grp * 256 + d)])
            scores[pl.ds(grp * 16, 16)] = acc
            return carry

        lax.fori_loop(0, CHUNK_SCORES // 16, group_body, 0)
        pltpu.sync_copy(
            scores,
            scores_out.at[pl.ds(base * SCORES_PER_ITEM + c * CHUNK_SCORES,
                                CHUNK_SCORES)])


@functools.partial(
    pl.kernel,
    out_type=jax.ShapeDtypeStruct((BATCH * SCORES_PER_ITEM,), jnp.float32),
    mesh=plsc.VectorSubcoreMesh(core_axis_name="c", subcore_axis_name="s"),
    compiler_params=pltpu.CompilerParams(needs_layout_passes=False,
                                         use_tc_tiling_on_sc=False),
    scratch_types=[
        pltpu.VMEM((B_PER_W,), jnp.int32),
        pltpu.VMEM((B_PER_W,), jnp.int32),
        pltpu.VMEM((B_PER_W * K_NEG,), jnp.int32),
        pltpu.VMEM((CHUNK, EMBED_DIM), jnp.float32),
        pltpu.VMEM((CHUNK, EMBED_DIM), jnp.float32),
        pltpu.VMEM((CHUNK * K_NEG, EMBED_DIM), jnp.float32),
        pltpu.VMEM((CHUNK_SCORES * 16,), jnp.float32),
        pltpu.VMEM((CHUNK_SCORES,), jnp.float32),
        pltpu.SemaphoreType.DMA,
    ],
)
def _sc_scores(*args):
    _sc_scores_kernel(*args)


def _tc_loss_kernel(s_ref, o_ref):
    x = s_ref[...]
    y = -jnp.log(jax.nn.sigmoid(x) + 1e-10)
    o_ref[0, 0] = jnp.sum(y) / BATCH


def kernel(center_words, context_words, negative_samples, center_emb,
           context_emb):
    cen_w = center_words.astype(jnp.int32)
    ctx_w = context_words.astype(jnp.int32)
    neg_w = negative_samples.astype(jnp.int32).reshape(-1)
    scores = _sc_scores(cen_w, ctx_w, neg_w, center_emb, context_emb)
    scores2d = scores.reshape(BATCH * SCORES_PER_ITEM // 128, 128)
    loss = pl.pallas_call(
        _tc_loss_kernel,
        out_shape=jax.ShapeDtypeStruct((1, 1), jnp.float32),
        in_specs=[pl.BlockSpec(memory_space=pltpu.VMEM)],
        out_specs=pl.BlockSpec(memory_space=pltpu.SMEM),
    )(scores2d)
    return loss[0, 0]
